# baseline (speedup 1.0000x reference)
import functools

import numpy as np
import jax
import jax.numpy as jnp
from jax import lax
from jax.experimental import pallas as pl
from jax.experimental.pallas import tpu as pltpu
from jax.experimental.pallas import tpu_sc as plsc

N = 10000
NP = 10240
E = 320000
C = 128
OC = 128
HD = 64
S = 1024
R = 100000
MASK_RATE = 0.25

NC = 2
NS = 16
NW = NC * NS

SB = E // 512
ROWS_PER_W = 320


def _aug_pidx2d():
    r = jax.random.uniform(jax.random.key(42), (N, C))
    perm = jnp.argsort(r, axis=0).astype(jnp.int32)
    mask = jax.random.uniform(jax.random.key(43), (N, C)) < MASK_RATE
    rows = jnp.arange(N, dtype=jnp.int32)[:, None]
    src_row = jnp.where(mask, perm, rows)
    cols = jnp.arange(C, dtype=jnp.int32)[None, :]
    return src_row * C + cols


def _precompute_pidx2d():
    try:
        try:
            dev = jax.devices("cpu")[0]
        except Exception:
            dev = None
        if dev is not None:
            with jax.default_device(dev):
                return np.asarray(_aug_pidx2d())
        return np.asarray(_aug_pidx2d())
    except Exception:
        return None


_PIDX2D = _precompute_pidx2d()
_FREQS = np.exp(np.linspace(0.0, 4.0, C)).astype(np.float32)

_SC_MESH = plsc.VectorSubcoreMesh(
    core_axis_name="c", subcore_axis_name="s", num_cores=NC, num_subcores=NS)


@functools.partial(
    pl.kernel,
    out_type=[
        jax.ShapeDtypeStruct((N * C,), jnp.float32),
        jax.ShapeDtypeStruct((N,), jnp.float32),
        jax.ShapeDtypeStruct((N, C), jnp.float32),
    ],
    mesh=_SC_MESH,
    scratch_types=[
        pltpu.VMEM((160, 128), jnp.int32),
        pltpu.VMEM((20480,), jnp.float32),
        pltpu.VMEM((320,), jnp.int32),
        pltpu.VMEM((320,), jnp.float32),
        pltpu.VMEM((160,), jnp.int32),
        pltpu.VMEM((160, C), jnp.float32),
        pltpu.SemaphoreType.DMA,
    ],
)
def _sc_pre(xf, pidx2d, seedt, bids, nids, emb,
            augf_o, seedg_o, shal_o,
            aidx_v, abuf_v, sidx_v, sbuf_v, nidx_v, ebuf_v, sem):
    c = lax.axis_index("c")
    s = lax.axis_index("s")
    w = c * NS + s
    r0 = jnp.minimum(ROWS_PER_W * w, N - ROWS_PER_W)

    for p in range(2):
        pltpu.sync_copy(pidx2d.at[pl.ds(r0 + 160 * p, 160)], aidx_v)

        def agrp(t, carry):
            cps = [
                pltpu.async_copy(
                    xf.at[aidx_v.at[8 * t + j]],
                    abuf_v.at[pl.ds((8 * t + j) * 128, 128)], sem)
                for j in range(8)
            ]
            for cp in cps:
                cp.wait()
            return carry

        lax.fori_loop(0, 20, agrp, 0)
        pltpu.sync_copy(abuf_v, augf_o.at[pl.ds((r0 + 160 * p) * C, 20480)])

    pltpu.sync_copy(bids.at[pl.ds(r0, 320)], sidx_v)
    scps = [
        pltpu.async_copy(seedt.at[sidx_v.at[pl.ds(16 * j, 16)]],
                         sbuf_v.at[pl.ds(16 * j, 16)], sem)
        for j in range(20)
    ]
    for cp in scps:
        cp.wait()
    pltpu.sync_copy(sbuf_v, seedg_o.at[pl.ds(r0, 320)])

    for p in range(2):
        pltpu.sync_copy(nids.at[pl.ds(r0 + 160 * p, 160)], nidx_v)
        ecps = [
            pltpu.async_copy(emb.at[nidx_v.at[pl.ds(16 * j, 16)]],
                             ebuf_v.at[pl.ds(16 * j, 16)], sem)
            for j in range(10)
        ]
        for cp in ecps:
            cp.wait()
        pltpu.sync_copy(ebuf_v, shal_o.at[pl.ds(r0 + 160 * p, 160)])


def _b_body(x_b, aug_b, sg_b, nt_b, shal_b, wenc, benc, wtime, btime, freqs_b,
            hpre_o, augpre_o):
    wenc_m = wenc[...]
    base = jnp.dot(x_b[...], wenc_m, preferred_element_type=jnp.float32)
    aug = jnp.dot(aug_b[...], wenc_m, preferred_element_type=jnp.float32)
    rel = sg_b[...] - nt_b[...]
    feats = jnp.cos(rel * freqs_b[...])
    tfeat = jnp.dot(feats, wtime[...], preferred_element_type=jnp.float32)
    add = tfeat + benc[...] + btime[...] + shal_b[...]
    hpre_o[...] = base + add
    augpre_o[...] = aug + add


def _tc_pre(x, aug, seedg, ntime, shallow, wenc, benc, wtime, btime):
    blk = N // 10
    return pl.pallas_call(
        _b_body,
        grid=(10,),
        in_specs=[
            pl.BlockSpec((blk, C), lambda i: (i, 0)),
            pl.BlockSpec((blk, C), lambda i: (i, 0)),
            pl.BlockSpec((blk, 1), lambda i: (i, 0)),
            pl.BlockSpec((blk, 1), lambda i: (i, 0)),
            pl.BlockSpec((blk, C), lambda i: (i, 0)),
            pl.BlockSpec((C, C), lambda i: (0, 0)),
            pl.BlockSpec((1, C), lambda i: (0, 0)),
            pl.BlockSpec((C, C), lambda i: (0, 0)),
            pl.BlockSpec((1, C), lambda i: (0, 0)),
            pl.BlockSpec((1, C), lambda i: (0, 0)),
        ],
        out_specs=[
            pl.BlockSpec((blk, C), lambda i: (i, 0)),
            pl.BlockSpec((blk, C), lambda i: (i, 0)),
        ],
        out_shape=[
            jax.ShapeDtypeStruct((N, C), jnp.float32),
            jax.ShapeDtypeStruct((N, C), jnp.float32),
        ],
    )(x, aug, seedg, ntime, shallow, wenc, benc, wtime, btime,
      jnp.asarray(_FREQS).reshape(1, C))


BLKS = 2560
EP = BLKS * 128
CHKS = BLKS // 16
CPT = CHKS // NS
RPT = NP // NS


@functools.partial(
    pl.kernel,
    out_type=[
        jax.ShapeDtypeStruct((NP, C), jnp.float32),
        jax.ShapeDtypeStruct((NP, C), jnp.float32),
        jax.ShapeDtypeStruct((NP,), jnp.float32),
    ],
    mesh=_SC_MESH,
    scratch_types=[
        pltpu.VMEM((128, C), jnp.float32),
        pltpu.VMEM((128, C), jnp.float32),
        pltpu.VMEM((16, 128), jnp.int32),
        pltpu.VMEM((16, 128), jnp.int32),
        pltpu.VMEM((128,), jnp.float32),
        pltpu.VMEM((RPT,), jnp.float32),
        pltpu.VMEM_SHARED((NP, C), jnp.float32),
        pltpu.VMEM_SHARED((NP,), jnp.float32),
        pltpu.SemaphoreType.DMA,
        pltpu.SemaphoreType.DMA,
    ],
)
def _sc_agg(hpre, augpre, src2d, dst2d,
            aggh_o, aggaug_o, cnt_o,
            rows0_v, rows1_v, sidx_v, didx_v, ones_v, cbuf_v,
            agg_sh, cnt_sh, sem0, sem1):
    c = lax.axis_index("c")
    s = lax.axis_index("s")

    def zrow(i, carry):
        def zj(j, inner):
            rows0_v[i, pl.ds(16 * j, 16)] = jnp.zeros((16,), jnp.float32)
            return inner
        return lax.fori_loop(0, C // 16, zj, carry)

    lax.fori_loop(0, 128, zrow, 0)

    def zcb(i, carry):
        cbuf_v[pl.ds(16 * i, 16)] = jnp.zeros((16,), jnp.float32)
        return carry

    lax.fori_loop(0, RPT // 16, zcb, 0)

    def of(i, carry):
        ones_v[pl.ds(16 * i, 16)] = jnp.ones((16,), jnp.float32)
        return carry

    lax.fori_loop(0, 8, of, 0)

    for k in range(RPT // 128):
        pltpu.sync_copy(rows0_v, agg_sh.at[pl.ds(RPT * s + 128 * k, 128)])
    pltpu.sync_copy(cbuf_v, cnt_sh.at[pl.ds(RPT * s, RPT)])
    plsc.subcore_barrier()

    def _edges(tbl, do_cnt):
        def estep(t, carry):
            q = s + NS * t
            pltpu.sync_copy(src2d.at[pl.ds(16 * q, 16)], sidx_v)
            pltpu.sync_copy(dst2d.at[pl.ds(16 * q, 16)], didx_v)
            bufs = (rows0_v, rows1_v)
            sems = (sem0, sem1)
            cp_prev = pltpu.async_copy(tbl.at[sidx_v.at[0]], rows0_v, sem0)
            for j in range(16):
                if j + 1 < 16:
                    cp_next = pltpu.async_copy(
                        tbl.at[sidx_v.at[j + 1]],
                        bufs[(j + 1) % 2], sems[(j + 1) % 2])
                cp_prev.wait()
                pltpu.sync_copy(bufs[j % 2], agg_sh.at[didx_v.at[j]],
                                add=True)
                if do_cnt:
                    pltpu.sync_copy(ones_v, cnt_sh.at[didx_v.at[j]],
                                    add=True)
                if j + 1 < 16:
                    cp_prev = cp_next
            return carry

        lax.fori_loop(0, CPT, estep, 0)

    @pl.when(c == 0)
    def _ch0():
        _edges(hpre, True)

    @pl.when(c == 1)
    def _ch1():
        _edges(augpre, False)

    plsc.subcore_barrier()

    def _write_agg(out):
        for k in range(RPT // 128):
            buf = rows0_v if k % 2 == 0 else rows1_v
            pltpu.sync_copy(agg_sh.at[pl.ds(RPT * s + 128 * k, 128)], buf)
            pltpu.sync_copy(buf, out.at[pl.ds(RPT * s + 128 * k, 128)])

    @pl.when(c == 0)
    def _w0():
        _write_agg(aggh_o)
        pltpu.sync_copy(cnt_sh.at[pl.ds(RPT * s, RPT)], cbuf_v)
        pltpu.sync_copy(cbuf_v, cnt_o.at[pl.ds(RPT * s, RPT)])

    @pl.when(c == 1)
    def _w1():
        _write_agg(aggaug_o)


def _d_body(hp, ap, ah, aa, cnt, wself, wneigh, bgnn, whead, bhead,
            wpred, bpred, out_o, augproj_o, hproj_o):
    inv = 1.0 / jnp.maximum(cnt[...], 1.0)
    ws = wself[...]
    wn = wneigh[...]
    hg = jnp.maximum(
        jnp.dot(hp[...], ws, preferred_element_type=jnp.float32)
        + jnp.dot(ah[...] * inv, wn, preferred_element_type=jnp.float32)
        + bgnn[...], 0.0)
    ag = jnp.maximum(
        jnp.dot(ap[...], ws, preferred_element_type=jnp.float32)
        + jnp.dot(aa[...] * inv, wn, preferred_element_type=jnp.float32)
        + bgnn[...], 0.0)
    out_o[...] = jnp.dot(ag, wpred[...],
                         preferred_element_type=jnp.float32) + bpred[...]
    augproj_o[...] = jnp.dot(ag, whead[...],
                             preferred_element_type=jnp.float32) + bhead[...]
    hproj_o[...] = jnp.dot(hg, whead[...],
                           preferred_element_type=jnp.float32) + bhead[...]


def _tc_post(hpre, augpre, aggh, aggaug, cnt, wself, wneigh, bgnn2,
             whead, bhead2, wpred, bpred2):
    blk = N // 10
    return pl.pallas_call(
        _d_body,
        grid=(10,),
        in_specs=[
            pl.BlockSpec((blk, C), lambda i: (i, 0)),
            pl.BlockSpec((blk, C), lambda i: (i, 0)),
            pl.BlockSpec((blk, C), lambda i: (i, 0)),
            pl.BlockSpec((blk, C), lambda i: (i, 0)),
            pl.BlockSpec((blk, 1), lambda i: (i, 0)),
            pl.BlockSpec((C, C), lambda i: (0, 0)),
            pl.BlockSpec((C, C), lambda i: (0, 0)),
            pl.BlockSpec((1, C), lambda i: (0, 0)),
            pl.BlockSpec((C, HD), lambda i: (0, 0)),
            pl.BlockSpec((1, HD), lambda i: (0, 0)),
            pl.BlockSpec((C, OC), lambda i: (0, 0)),
            pl.BlockSpec((1, OC), lambda i: (0, 0)),
        ],
        out_specs=[
            pl.BlockSpec((blk, OC), lambda i: (i, 0)),
            pl.BlockSpec((blk, HD), lambda i: (i, 0)),
            pl.BlockSpec((blk, HD), lambda i: (i, 0)),
        ],
        out_shape=[
            jax.ShapeDtypeStruct((N, OC), jnp.float32),
            jax.ShapeDtypeStruct((N, HD), jnp.float32),
            jax.ShapeDtypeStruct((N, HD), jnp.float32),
        ],
    )(hpre, augpre, aggh, aggaug, cnt, wself, wneigh, bgnn2, whead, bhead2,
      wpred, bpred2)


def kernel(x, edge_index, seed_time, node_time, batch_ids, n_id,
           W_enc, b_enc, W_time, b_time, emb_table,
           W_self, W_neigh, b_gnn, W_head, b_head, W_pred, b_pred):
    pidx2d = jnp.asarray(_PIDX2D) if _PIDX2D is not None else _aug_pidx2d()
    aug_f, seedg, shallow = _sc_pre(
        x.reshape(-1), pidx2d, seed_time, batch_ids, n_id, emb_table)
    h_pre, aug_pre = _tc_pre(
        x, aug_f.reshape(N, C), seedg.reshape(N, 1), node_time.reshape(N, 1),
        shallow, W_enc, b_enc.reshape(1, C), W_time, b_time.reshape(1, C))
    spad = jnp.zeros((EP - E,), jnp.int32)
    dpad = jnp.full((EP - E,), NP - 1, jnp.int32)
    src2d = jnp.concatenate([edge_index[0], spad]).reshape(BLKS, 128)
    dst2d = jnp.concatenate([edge_index[1], dpad]).reshape(BLKS, 128)
    agg_h, agg_aug, cnt = _sc_agg(h_pre, aug_pre, src2d, dst2d)
    return _tc_post(
        h_pre, aug_pre, agg_h[:N], agg_aug[:N], cnt[:N].reshape(N, 1),
        W_self, W_neigh,
        b_gnn.reshape(1, C), W_head, b_head.reshape(1, HD),
        W_pred, b_pred.reshape(1, OC))


# speedup vs baseline: 1.0005x; 1.0005x over previous
import functools

import numpy as np
import jax
import jax.numpy as jnp
from jax import lax
from jax.experimental import pallas as pl
from jax.experimental.pallas import tpu as pltpu
from jax.experimental.pallas import tpu_sc as plsc

N = 10000
NP = 10240
E = 320000
C = 128
OC = 128
HD = 64
S = 1024
R = 100000
MASK_RATE = 0.25

NC = 2
NS = 16
NW = NC * NS

SB = E // 512
ROWS_PER_W = 320


def _aug_pidx2d():
    r = jax.random.uniform(jax.random.key(42), (N, C))
    perm = jnp.argsort(r, axis=0).astype(jnp.int32)
    mask = jax.random.uniform(jax.random.key(43), (N, C)) < MASK_RATE
    rows = jnp.arange(N, dtype=jnp.int32)[:, None]
    src_row = jnp.where(mask, perm, rows)
    cols = jnp.arange(C, dtype=jnp.int32)[None, :]
    return src_row * C + cols


def _precompute_pidx2d():
    try:
        try:
            dev = jax.devices("cpu")[0]
        except Exception:
            dev = None
        if dev is not None:
            with jax.default_device(dev):
                return np.asarray(_aug_pidx2d())
        return np.asarray(_aug_pidx2d())
    except Exception:
        return None


_PIDX2D = _precompute_pidx2d()
_FREQS = np.exp(np.linspace(0.0, 4.0, C)).astype(np.float32)

_SC_MESH = plsc.VectorSubcoreMesh(
    core_axis_name="c", subcore_axis_name="s", num_cores=NC, num_subcores=NS)


@functools.partial(
    pl.kernel,
    out_type=[
        jax.ShapeDtypeStruct((N * C,), jnp.float32),
        jax.ShapeDtypeStruct((N,), jnp.float32),
        jax.ShapeDtypeStruct((N, C), jnp.float32),
    ],
    mesh=_SC_MESH,
    scratch_types=[
        pltpu.VMEM((160, 128), jnp.int32),
        pltpu.VMEM((20480,), jnp.float32),
        pltpu.VMEM((320,), jnp.int32),
        pltpu.VMEM((320,), jnp.float32),
        pltpu.VMEM((160,), jnp.int32),
        pltpu.VMEM((160, C), jnp.float32),
        pltpu.SemaphoreType.DMA,
    ],
)
def _sc_pre(xf, pidx2d, seedt, bids, nids, emb,
            augf_o, seedg_o, shal_o,
            aidx_v, abuf_v, sidx_v, sbuf_v, nidx_v, ebuf_v, sem):
    c = lax.axis_index("c")
    s = lax.axis_index("s")
    w = c * NS + s
    r0 = jnp.minimum(ROWS_PER_W * w, N - ROWS_PER_W)

    for p in range(2):
        pltpu.sync_copy(pidx2d.at[pl.ds(r0 + 160 * p, 160)], aidx_v)

        def agrp(t, carry):
            cps = [
                pltpu.async_copy(
                    xf.at[aidx_v.at[8 * t + j]],
                    abuf_v.at[pl.ds((8 * t + j) * 128, 128)], sem)
                for j in range(8)
            ]
            for cp in cps:
                cp.wait()
            return carry

        lax.fori_loop(0, 20, agrp, 0)
        pltpu.sync_copy(abuf_v, augf_o.at[pl.ds((r0 + 160 * p) * C, 20480)])

    pltpu.sync_copy(bids.at[pl.ds(r0, 320)], sidx_v)
    scps = [
        pltpu.async_copy(seedt.at[sidx_v.at[pl.ds(16 * j, 16)]],
                         sbuf_v.at[pl.ds(16 * j, 16)], sem)
        for j in range(20)
    ]
    for cp in scps:
        cp.wait()
    pltpu.sync_copy(sbuf_v, seedg_o.at[pl.ds(r0, 320)])

    for p in range(2):
        pltpu.sync_copy(nids.at[pl.ds(r0 + 160 * p, 160)], nidx_v)
        ecps = [
            pltpu.async_copy(emb.at[nidx_v.at[pl.ds(16 * j, 16)]],
                             ebuf_v.at[pl.ds(16 * j, 16)], sem)
            for j in range(10)
        ]
        for cp in ecps:
            cp.wait()
        pltpu.sync_copy(ebuf_v, shal_o.at[pl.ds(r0 + 160 * p, 160)])


def _b_body(x_b, aug_b, sg_b, nt_b, shal_b, wenc, benc, wtime, btime, freqs_b,
            hpre_o, augpre_o):
    wenc_m = wenc[...]
    base = jnp.dot(x_b[...], wenc_m, preferred_element_type=jnp.float32)
    aug = jnp.dot(aug_b[...], wenc_m, preferred_element_type=jnp.float32)
    rel = sg_b[...] - nt_b[...]
    feats = jnp.cos(rel * freqs_b[...])
    tfeat = jnp.dot(feats, wtime[...], preferred_element_type=jnp.float32)
    add = tfeat + benc[...] + btime[...] + shal_b[...]
    hpre_o[...] = base + add
    augpre_o[...] = aug + add


def _tc_pre(x, aug, seedg, ntime, shallow, wenc, benc, wtime, btime):
    blk = N // 10
    return pl.pallas_call(
        _b_body,
        grid=(10,),
        in_specs=[
            pl.BlockSpec((blk, C), lambda i: (i, 0)),
            pl.BlockSpec((blk, C), lambda i: (i, 0)),
            pl.BlockSpec((blk, 1), lambda i: (i, 0)),
            pl.BlockSpec((blk, 1), lambda i: (i, 0)),
            pl.BlockSpec((blk, C), lambda i: (i, 0)),
            pl.BlockSpec((C, C), lambda i: (0, 0)),
            pl.BlockSpec((1, C), lambda i: (0, 0)),
            pl.BlockSpec((C, C), lambda i: (0, 0)),
            pl.BlockSpec((1, C), lambda i: (0, 0)),
            pl.BlockSpec((1, C), lambda i: (0, 0)),
        ],
        out_specs=[
            pl.BlockSpec((blk, C), lambda i: (i, 0)),
            pl.BlockSpec((blk, C), lambda i: (i, 0)),
        ],
        out_shape=[
            jax.ShapeDtypeStruct((N, C), jnp.float32),
            jax.ShapeDtypeStruct((N, C), jnp.float32),
        ],
    )(x, aug, seedg, ntime, shallow, wenc, benc, wtime, btime,
      jnp.asarray(_FREQS).reshape(1, C))


BLKS = 2560
EP = BLKS * 128
CHKS = BLKS // 16
CPT = CHKS // NS
RPT = NP // NS


@functools.partial(
    pl.kernel,
    out_type=[
        jax.ShapeDtypeStruct((NP, C), jnp.float32),
        jax.ShapeDtypeStruct((NP, C), jnp.float32),
        jax.ShapeDtypeStruct((NP,), jnp.float32),
    ],
    mesh=_SC_MESH,
    scratch_types=[
        pltpu.VMEM((128, C), jnp.float32),
        pltpu.VMEM((128, C), jnp.float32),
        pltpu.VMEM((16, 128), jnp.int32),
        pltpu.VMEM((16, 128), jnp.int32),
        pltpu.VMEM((128,), jnp.float32),
        pltpu.VMEM((RPT,), jnp.float32),
        pltpu.VMEM_SHARED((NP, C), jnp.float32),
        pltpu.VMEM_SHARED((NP,), jnp.float32),
        pltpu.SemaphoreType.DMA,
        pltpu.SemaphoreType.DMA,
        pltpu.SemaphoreType.DMA,
        pltpu.SemaphoreType.DMA,
        pltpu.SemaphoreType.DMA,
    ],
)
def _sc_agg(hpre, augpre, src2d, dst2d,
            aggh_o, aggaug_o, cnt_o,
            rows0_v, rows1_v, sidx_v, didx_v, ones_v, cbuf_v,
            agg_sh, cnt_sh, sem0, sem1, ssem0, ssem1, csem):
    c = lax.axis_index("c")
    s = lax.axis_index("s")

    def zrow(i, carry):
        def zj(j, inner):
            rows0_v[i, pl.ds(16 * j, 16)] = jnp.zeros((16,), jnp.float32)
            return inner
        return lax.fori_loop(0, C // 16, zj, carry)

    lax.fori_loop(0, 128, zrow, 0)

    def zcb(i, carry):
        cbuf_v[pl.ds(16 * i, 16)] = jnp.zeros((16,), jnp.float32)
        return carry

    lax.fori_loop(0, RPT // 16, zcb, 0)

    def of(i, carry):
        ones_v[pl.ds(16 * i, 16)] = jnp.ones((16,), jnp.float32)
        return carry

    lax.fori_loop(0, 8, of, 0)

    for k in range(RPT // 128):
        pltpu.sync_copy(rows0_v, agg_sh.at[pl.ds(RPT * s + 128 * k, 128)])
    pltpu.sync_copy(cbuf_v, cnt_sh.at[pl.ds(RPT * s, RPT)])
    plsc.subcore_barrier()

    def _edges(tbl, do_cnt):
        def estep(t, carry):
            q = s + NS * t
            pltpu.sync_copy(src2d.at[pl.ds(16 * q, 16)], sidx_v)
            pltpu.sync_copy(dst2d.at[pl.ds(16 * q, 16)], didx_v)
            bufs = (rows0_v, rows1_v)
            gsems = (sem0, sem1)
            ssems = (ssem0, ssem1)
            cp_prev = pltpu.async_copy(tbl.at[sidx_v.at[0]], rows0_v, sem0)
            sc_prev = [None, None]
            ccp = None
            for j in range(16):
                if j + 1 < 16:
                    b = (j + 1) % 2
                    if sc_prev[b] is not None:
                        sc_prev[b].wait()
                        sc_prev[b] = None
                    cp_next = pltpu.async_copy(
                        tbl.at[sidx_v.at[j + 1]], bufs[b], gsems[b])
                cp_prev.wait()
                sc_prev[j % 2] = pltpu.async_copy(
                    bufs[j % 2], agg_sh.at[didx_v.at[j]], ssems[j % 2],
                    add=True)
                if do_cnt:
                    if ccp is not None:
                        ccp.wait()
                    ccp = pltpu.async_copy(ones_v, cnt_sh.at[didx_v.at[j]],
                                           csem, add=True)
                if j + 1 < 16:
                    cp_prev = cp_next
            for b in range(2):
                if sc_prev[b] is not None:
                    sc_prev[b].wait()
            if ccp is not None:
                ccp.wait()
            return carry

        lax.fori_loop(0, CPT, estep, 0)

    @pl.when(c == 0)
    def _ch0():
        _edges(hpre, True)

    @pl.when(c == 1)
    def _ch1():
        _edges(augpre, False)

    plsc.subcore_barrier()

    def _write_agg(out):
        for k in range(RPT // 128):
            buf = rows0_v if k % 2 == 0 else rows1_v
            pltpu.sync_copy(agg_sh.at[pl.ds(RPT * s + 128 * k, 128)], buf)
            pltpu.sync_copy(buf, out.at[pl.ds(RPT * s + 128 * k, 128)])

    @pl.when(c == 0)
    def _w0():
        _write_agg(aggh_o)
        pltpu.sync_copy(cnt_sh.at[pl.ds(RPT * s, RPT)], cbuf_v)
        pltpu.sync_copy(cbuf_v, cnt_o.at[pl.ds(RPT * s, RPT)])

    @pl.when(c == 1)
    def _w1():
        _write_agg(aggaug_o)


def _d_body(hp, ap, ah, aa, cnt, wself, wneigh, bgnn, whead, bhead,
            wpred, bpred, out_o, augproj_o, hproj_o):
    inv = 1.0 / jnp.maximum(cnt[...], 1.0)
    ws = wself[...]
    wn = wneigh[...]
    hg = jnp.maximum(
        jnp.dot(hp[...], ws, preferred_element_type=jnp.float32)
        + jnp.dot(ah[...] * inv, wn, preferred_element_type=jnp.float32)
        + bgnn[...], 0.0)
    ag = jnp.maximum(
        jnp.dot(ap[...], ws, preferred_element_type=jnp.float32)
        + jnp.dot(aa[...] * inv, wn, preferred_element_type=jnp.float32)
        + bgnn[...], 0.0)
    out_o[...] = jnp.dot(ag, wpred[...],
                         preferred_element_type=jnp.float32) + bpred[...]
    augproj_o[...] = jnp.dot(ag, whead[...],
                             preferred_element_type=jnp.float32) + bhead[...]
    hproj_o[...] = jnp.dot(hg, whead[...],
                           preferred_element_type=jnp.float32) + bhead[...]


def _tc_post(hpre, augpre, aggh, aggaug, cnt, wself, wneigh, bgnn2,
             whead, bhead2, wpred, bpred2):
    blk = N // 10
    return pl.pallas_call(
        _d_body,
        grid=(10,),
        in_specs=[
            pl.BlockSpec((blk, C), lambda i: (i, 0)),
            pl.BlockSpec((blk, C), lambda i: (i, 0)),
            pl.BlockSpec((blk, C), lambda i: (i, 0)),
            pl.BlockSpec((blk, C), lambda i: (i, 0)),
            pl.BlockSpec((blk, 1), lambda i: (i, 0)),
            pl.BlockSpec((C, C), lambda i: (0, 0)),
            pl.BlockSpec((C, C), lambda i: (0, 0)),
            pl.BlockSpec((1, C), lambda i: (0, 0)),
            pl.BlockSpec((C, HD), lambda i: (0, 0)),
            pl.BlockSpec((1, HD), lambda i: (0, 0)),
            pl.BlockSpec((C, OC), lambda i: (0, 0)),
            pl.BlockSpec((1, OC), lambda i: (0, 0)),
        ],
        out_specs=[
            pl.BlockSpec((blk, OC), lambda i: (i, 0)),
            pl.BlockSpec((blk, HD), lambda i: (i, 0)),
            pl.BlockSpec((blk, HD), lambda i: (i, 0)),
        ],
        out_shape=[
            jax.ShapeDtypeStruct((N, OC), jnp.float32),
            jax.ShapeDtypeStruct((N, HD), jnp.float32),
            jax.ShapeDtypeStruct((N, HD), jnp.float32),
        ],
    )(hpre, augpre, aggh, aggaug, cnt, wself, wneigh, bgnn2, whead, bhead2,
      wpred, bpred2)


def kernel(x, edge_index, seed_time, node_time, batch_ids, n_id,
           W_enc, b_enc, W_time, b_time, emb_table,
           W_self, W_neigh, b_gnn, W_head, b_head, W_pred, b_pred):
    pidx2d = jnp.asarray(_PIDX2D) if _PIDX2D is not None else _aug_pidx2d()
    aug_f, seedg, shallow = _sc_pre(
        x.reshape(-1), pidx2d, seed_time, batch_ids, n_id, emb_table)
    h_pre, aug_pre = _tc_pre(
        x, aug_f.reshape(N, C), seedg.reshape(N, 1), node_time.reshape(N, 1),
        shallow, W_enc, b_enc.reshape(1, C), W_time, b_time.reshape(1, C))
    spad = jnp.zeros((EP - E,), jnp.int32)
    dpad = jnp.full((EP - E,), NP - 1, jnp.int32)
    src2d = jnp.concatenate([edge_index[0], spad]).reshape(BLKS, 128)
    dst2d = jnp.concatenate([edge_index[1], dpad]).reshape(BLKS, 128)
    agg_h, agg_aug, cnt = _sc_agg(h_pre, aug_pre, src2d, dst2d)
    return _tc_post(
        h_pre, aug_pre, agg_h[:N], agg_aug[:N], cnt[:N].reshape(N, 1),
        W_self, W_neigh,
        b_gnn.reshape(1, C), W_head, b_head.reshape(1, HD),
        W_pred, b_pred.reshape(1, OC))


# speedup vs baseline: 1.0027x; 1.0021x over previous
import functools

import numpy as np
import jax
import jax.numpy as jnp
from jax import lax
from jax.experimental import pallas as pl
from jax.experimental.pallas import tpu as pltpu
from jax.experimental.pallas import tpu_sc as plsc

N = 10000
NP = 10240
E = 320000
C = 128
OC = 128
HD = 64
S = 1024
R = 100000
MASK_RATE = 0.25

NC = 2
NS = 16
NW = NC * NS

SB = E // 512
ROWS_PER_W = 320


def _aug_pidx2d():
    r = jax.random.uniform(jax.random.key(42), (N, C))
    perm = jnp.argsort(r, axis=0).astype(jnp.int32)
    mask = jax.random.uniform(jax.random.key(43), (N, C)) < MASK_RATE
    rows = jnp.arange(N, dtype=jnp.int32)[:, None]
    src_row = jnp.where(mask, perm, rows)
    cols = jnp.arange(C, dtype=jnp.int32)[None, :]
    return src_row * C + cols


def _precompute_pidx2d():
    try:
        try:
            dev = jax.devices("cpu")[0]
        except Exception:
            dev = None
        if dev is not None:
            with jax.default_device(dev):
                return np.asarray(_aug_pidx2d())
        return np.asarray(_aug_pidx2d())
    except Exception:
        return None


_PIDX2D = _precompute_pidx2d()
_FREQS = np.exp(np.linspace(0.0, 4.0, C)).astype(np.float32)

_SC_MESH = plsc.VectorSubcoreMesh(
    core_axis_name="c", subcore_axis_name="s", num_cores=NC, num_subcores=NS)


@functools.partial(
    pl.kernel,
    out_type=[
        jax.ShapeDtypeStruct((N * C,), jnp.float32),
        jax.ShapeDtypeStruct((N,), jnp.float32),
        jax.ShapeDtypeStruct((N, C), jnp.float32),
    ],
    mesh=_SC_MESH,
    scratch_types=[
        pltpu.VMEM((160, 128), jnp.int32),
        pltpu.VMEM((20480,), jnp.float32),
        pltpu.VMEM((320,), jnp.int32),
        pltpu.VMEM((320,), jnp.float32),
        pltpu.VMEM((160,), jnp.int32),
        pltpu.VMEM((160, C), jnp.float32),
        pltpu.SemaphoreType.DMA,
    ],
)
def _sc_pre(xf, pidx2d, seedt, bids, nids, emb,
            augf_o, seedg_o, shal_o,
            aidx_v, abuf_v, sidx_v, sbuf_v, nidx_v, ebuf_v, sem):
    c = lax.axis_index("c")
    s = lax.axis_index("s")
    w = c * NS + s
    r0 = jnp.minimum(ROWS_PER_W * w, N - ROWS_PER_W)

    for p in range(2):
        pltpu.sync_copy(pidx2d.at[pl.ds(r0 + 160 * p, 160)], aidx_v)

        def agrp(t, carry):
            cps = [
                pltpu.async_copy(
                    xf.at[aidx_v.at[8 * t + j]],
                    abuf_v.at[pl.ds((8 * t + j) * 128, 128)], sem)
                for j in range(8)
            ]
            for cp in cps:
                cp.wait()
            return carry

        lax.fori_loop(0, 20, agrp, 0)
        pltpu.sync_copy(abuf_v, augf_o.at[pl.ds((r0 + 160 * p) * C, 20480)])

    pltpu.sync_copy(bids.at[pl.ds(r0, 320)], sidx_v)
    scps = [
        pltpu.async_copy(seedt.at[sidx_v.at[pl.ds(16 * j, 16)]],
                         sbuf_v.at[pl.ds(16 * j, 16)], sem)
        for j in range(20)
    ]
    for cp in scps:
        cp.wait()
    pltpu.sync_copy(sbuf_v, seedg_o.at[pl.ds(r0, 320)])

    for p in range(2):
        pltpu.sync_copy(nids.at[pl.ds(r0 + 160 * p, 160)], nidx_v)
        ecps = [
            pltpu.async_copy(emb.at[nidx_v.at[pl.ds(16 * j, 16)]],
                             ebuf_v.at[pl.ds(16 * j, 16)], sem)
            for j in range(10)
        ]
        for cp in ecps:
            cp.wait()
        pltpu.sync_copy(ebuf_v, shal_o.at[pl.ds(r0 + 160 * p, 160)])


def _b_body(x_b, aug_b, sg_b, nt_b, shal_b, wenc, benc, wtime, btime, freqs_b,
            hpre_o, augpre_o):
    wenc_m = wenc[...]
    base = jnp.dot(x_b[...], wenc_m, preferred_element_type=jnp.float32)
    aug = jnp.dot(aug_b[...], wenc_m, preferred_element_type=jnp.float32)
    rel = sg_b[...] - nt_b[...]
    feats = jnp.cos(rel * freqs_b[...])
    tfeat = jnp.dot(feats, wtime[...], preferred_element_type=jnp.float32)
    add = tfeat + benc[...] + btime[...] + shal_b[...]
    hpre_o[...] = base + add
    augpre_o[...] = aug + add


def _tc_pre(x, aug, seedg, ntime, shallow, wenc, benc, wtime, btime):
    blk = N // 10
    return pl.pallas_call(
        _b_body,
        grid=(10,),
        in_specs=[
            pl.BlockSpec((blk, C), lambda i: (i, 0)),
            pl.BlockSpec((blk, C), lambda i: (i, 0)),
            pl.BlockSpec((blk, 1), lambda i: (i, 0)),
            pl.BlockSpec((blk, 1), lambda i: (i, 0)),
            pl.BlockSpec((blk, C), lambda i: (i, 0)),
            pl.BlockSpec((C, C), lambda i: (0, 0)),
            pl.BlockSpec((1, C), lambda i: (0, 0)),
            pl.BlockSpec((C, C), lambda i: (0, 0)),
            pl.BlockSpec((1, C), lambda i: (0, 0)),
            pl.BlockSpec((1, C), lambda i: (0, 0)),
        ],
        out_specs=[
            pl.BlockSpec((blk, C), lambda i: (i, 0)),
            pl.BlockSpec((blk, C), lambda i: (i, 0)),
        ],
        out_shape=[
            jax.ShapeDtypeStruct((N, C), jnp.float32),
            jax.ShapeDtypeStruct((N, C), jnp.float32),
        ],
    )(x, aug, seedg, ntime, shallow, wenc, benc, wtime, btime,
      jnp.asarray(_FREQS).reshape(1, C))


BLKS = 2560
EP = BLKS * 128
CHKS = BLKS // 16
CPT = CHKS // NS
RPT = NP // NS


@functools.partial(
    pl.kernel,
    out_type=[
        jax.ShapeDtypeStruct((NP, C), jnp.float32),
        jax.ShapeDtypeStruct((NP, C), jnp.float32),
        jax.ShapeDtypeStruct((NP,), jnp.float32),
    ],
    mesh=_SC_MESH,
    scratch_types=[
        pltpu.VMEM((128, C), jnp.float32),
        pltpu.VMEM((128, C), jnp.float32),
        pltpu.VMEM((16, 128), jnp.int32),
        pltpu.VMEM((16, 128), jnp.int32),
        pltpu.VMEM((128,), jnp.float32),
        pltpu.VMEM((RPT,), jnp.float32),
        pltpu.VMEM_SHARED((NP, C), jnp.float32),
        pltpu.VMEM_SHARED((NP,), jnp.float32),
        pltpu.SemaphoreType.DMA,
        pltpu.SemaphoreType.DMA,
        pltpu.SemaphoreType.DMA,
        pltpu.SemaphoreType.DMA,
        pltpu.SemaphoreType.DMA,
    ],
)
def _sc_agg(hpre, augpre, src2d, dst2d,
            aggh_o, aggaug_o, cnt_o,
            rows0_v, rows1_v, sidx_v, didx_v, ones_v, cbuf_v,
            agg_sh, cnt_sh, sem0, sem1, ssem0, ssem1, csem):
    c = lax.axis_index("c")
    s = lax.axis_index("s")

    def zrow(i, carry):
        def zj(j, inner):
            rows0_v[i, pl.ds(16 * j, 16)] = jnp.zeros((16,), jnp.float32)
            return inner
        return lax.fori_loop(0, C // 16, zj, carry)

    lax.fori_loop(0, 128, zrow, 0)

    def zcb(i, carry):
        cbuf_v[pl.ds(16 * i, 16)] = jnp.zeros((16,), jnp.float32)
        return carry

    lax.fori_loop(0, RPT // 16, zcb, 0)

    def of(i, carry):
        ones_v[pl.ds(16 * i, 16)] = jnp.ones((16,), jnp.float32)
        return carry

    lax.fori_loop(0, 8, of, 0)

    for k in range(RPT // 128):
        pltpu.sync_copy(rows0_v, agg_sh.at[pl.ds(RPT * s + 128 * k, 128)])
    pltpu.sync_copy(cbuf_v, cnt_sh.at[pl.ds(RPT * s, RPT)])
    plsc.subcore_barrier()

    def _edges(tbl, do_cnt):
        def estep(t, carry):
            q = s + NS * t
            pltpu.sync_copy(src2d.at[pl.ds(16 * q, 16)], sidx_v)
            pltpu.sync_copy(dst2d.at[pl.ds(16 * q, 16)], didx_v)
            bufs = (rows0_v, rows1_v)
            gsems = (sem0, sem1)
            ssems = (ssem0, ssem1)
            def _gather2(j, b):
                return [
                    pltpu.async_copy(
                        tbl.at[sidx_v.at[j, pl.ds(64 * h, 64)]],
                        bufs[b].at[pl.ds(64 * h, 64)], gsems[b])
                    for h in range(2)
                ]

            cp_prev = _gather2(0, 0)
            sc_prev = [None, None]
            ccp = None
            for j in range(16):
                if j + 1 < 16:
                    b = (j + 1) % 2
                    if sc_prev[b] is not None:
                        sc_prev[b].wait()
                        sc_prev[b] = None
                    cp_next = _gather2(j + 1, b)
                for cp in cp_prev:
                    cp.wait()
                sc_prev[j % 2] = pltpu.async_copy(
                    bufs[j % 2], agg_sh.at[didx_v.at[j]], ssems[j % 2],
                    add=True)
                if do_cnt:
                    if ccp is not None:
                        ccp.wait()
                    ccp = pltpu.async_copy(ones_v, cnt_sh.at[didx_v.at[j]],
                                           csem, add=True)
                if j + 1 < 16:
                    cp_prev = cp_next
            for b in range(2):
                if sc_prev[b] is not None:
                    sc_prev[b].wait()
            if ccp is not None:
                ccp.wait()
            return carry

        lax.fori_loop(0, CPT, estep, 0)

    @pl.when(c == 0)
    def _ch0():
        _edges(hpre, True)

    @pl.when(c == 1)
    def _ch1():
        _edges(augpre, False)

    plsc.subcore_barrier()

    def _write_agg(out):
        for k in range(RPT // 128):
            buf = rows0_v if k % 2 == 0 else rows1_v
            pltpu.sync_copy(agg_sh.at[pl.ds(RPT * s + 128 * k, 128)], buf)
            pltpu.sync_copy(buf, out.at[pl.ds(RPT * s + 128 * k, 128)])

    @pl.when(c == 0)
    def _w0():
        _write_agg(aggh_o)
        pltpu.sync_copy(cnt_sh.at[pl.ds(RPT * s, RPT)], cbuf_v)
        pltpu.sync_copy(cbuf_v, cnt_o.at[pl.ds(RPT * s, RPT)])

    @pl.when(c == 1)
    def _w1():
        _write_agg(aggaug_o)


def _d_body(hp, ap, ah, aa, cnt, wself, wneigh, bgnn, whead, bhead,
            wpred, bpred, out_o, augproj_o, hproj_o):
    inv = 1.0 / jnp.maximum(cnt[...], 1.0)
    ws = wself[...]
    wn = wneigh[...]
    hg = jnp.maximum(
        jnp.dot(hp[...], ws, preferred_element_type=jnp.float32)
        + jnp.dot(ah[...] * inv, wn, preferred_element_type=jnp.float32)
        + bgnn[...], 0.0)
    ag = jnp.maximum(
        jnp.dot(ap[...], ws, preferred_element_type=jnp.float32)
        + jnp.dot(aa[...] * inv, wn, preferred_element_type=jnp.float32)
        + bgnn[...], 0.0)
    out_o[...] = jnp.dot(ag, wpred[...],
                         preferred_element_type=jnp.float32) + bpred[...]
    augproj_o[...] = jnp.dot(ag, whead[...],
                             preferred_element_type=jnp.float32) + bhead[...]
    hproj_o[...] = jnp.dot(hg, whead[...],
                           preferred_element_type=jnp.float32) + bhead[...]


def _tc_post(hpre, augpre, aggh, aggaug, cnt, wself, wneigh, bgnn2,
             whead, bhead2, wpred, bpred2):
    blk = N // 10
    return pl.pallas_call(
        _d_body,
        grid=(10,),
        in_specs=[
            pl.BlockSpec((blk, C), lambda i: (i, 0)),
            pl.BlockSpec((blk, C), lambda i: (i, 0)),
            pl.BlockSpec((blk, C), lambda i: (i, 0)),
            pl.BlockSpec((blk, C), lambda i: (i, 0)),
            pl.BlockSpec((blk, 1), lambda i: (i, 0)),
            pl.BlockSpec((C, C), lambda i: (0, 0)),
            pl.BlockSpec((C, C), lambda i: (0, 0)),
            pl.BlockSpec((1, C), lambda i: (0, 0)),
            pl.BlockSpec((C, HD), lambda i: (0, 0)),
            pl.BlockSpec((1, HD), lambda i: (0, 0)),
            pl.BlockSpec((C, OC), lambda i: (0, 0)),
            pl.BlockSpec((1, OC), lambda i: (0, 0)),
        ],
        out_specs=[
            pl.BlockSpec((blk, OC), lambda i: (i, 0)),
            pl.BlockSpec((blk, HD), lambda i: (i, 0)),
            pl.BlockSpec((blk, HD), lambda i: (i, 0)),
        ],
        out_shape=[
            jax.ShapeDtypeStruct((N, OC), jnp.float32),
            jax.ShapeDtypeStruct((N, HD), jnp.float32),
            jax.ShapeDtypeStruct((N, HD), jnp.float32),
        ],
    )(hpre, augpre, aggh, aggaug, cnt, wself, wneigh, bgnn2, whead, bhead2,
      wpred, bpred2)


def kernel(x, edge_index, seed_time, node_time, batch_ids, n_id,
           W_enc, b_enc, W_time, b_time, emb_table,
           W_self, W_neigh, b_gnn, W_head, b_head, W_pred, b_pred):
    pidx2d = jnp.asarray(_PIDX2D) if _PIDX2D is not None else _aug_pidx2d()
    aug_f, seedg, shallow = _sc_pre(
        x.reshape(-1), pidx2d, seed_time, batch_ids, n_id, emb_table)
    h_pre, aug_pre = _tc_pre(
        x, aug_f.reshape(N, C), seedg.reshape(N, 1), node_time.reshape(N, 1),
        shallow, W_enc, b_enc.reshape(1, C), W_time, b_time.reshape(1, C))
    spad = jnp.zeros((EP - E,), jnp.int32)
    dpad = jnp.full((EP - E,), NP - 1, jnp.int32)
    src2d = jnp.concatenate([edge_index[0], spad]).reshape(BLKS, 128)
    dst2d = jnp.concatenate([edge_index[1], dpad]).reshape(BLKS, 128)
    agg_h, agg_aug, cnt = _sc_agg(h_pre, aug_pre, src2d, dst2d)
    return _tc_post(
        h_pre, aug_pre, agg_h[:N], agg_aug[:N], cnt[:N].reshape(N, 1),
        W_self, W_neigh,
        b_gnn.reshape(1, C), W_head, b_head.reshape(1, HD),
        W_pred, b_pred.reshape(1, OC))


# speedup vs baseline: 1.0374x; 1.0347x over previous
import functools

import numpy as np
import jax
import jax.numpy as jnp
from jax import lax
from jax.experimental import pallas as pl
from jax.experimental.pallas import tpu as pltpu
from jax.experimental.pallas import tpu_sc as plsc

N = 10000
NP = 10240
E = 320000
C = 128
OC = 128
HD = 64
S = 1024
R = 100000
MASK_RATE = 0.25

NC = 2
NS = 16
NW = NC * NS

SB = E // 512
ROWS_PER_W = 320


def _aug_pidx2d():
    r = jax.random.uniform(jax.random.key(42), (N, C))
    perm = jnp.argsort(r, axis=0).astype(jnp.int32)
    mask = jax.random.uniform(jax.random.key(43), (N, C)) < MASK_RATE
    rows = jnp.arange(N, dtype=jnp.int32)[:, None]
    src_row = jnp.where(mask, perm, rows)
    cols = jnp.arange(C, dtype=jnp.int32)[None, :]
    return src_row * C + cols


def _precompute_pidx2d():
    try:
        try:
            dev = jax.devices("cpu")[0]
        except Exception:
            dev = None
        if dev is not None:
            with jax.default_device(dev):
                return np.asarray(_aug_pidx2d())
        return np.asarray(_aug_pidx2d())
    except Exception:
        return None


_PIDX2D = _precompute_pidx2d()
_FREQS = np.exp(np.linspace(0.0, 4.0, C)).astype(np.float32)

_SC_MESH = plsc.VectorSubcoreMesh(
    core_axis_name="c", subcore_axis_name="s", num_cores=NC, num_subcores=NS)


@functools.partial(
    pl.kernel,
    out_type=[
        jax.ShapeDtypeStruct((N * C,), jnp.float32),
        jax.ShapeDtypeStruct((N,), jnp.float32),
        jax.ShapeDtypeStruct((N, C), jnp.float32),
    ],
    mesh=_SC_MESH,
    scratch_types=[
        pltpu.VMEM((20480,), jnp.int32),
        pltpu.VMEM((20480,), jnp.float32),
        pltpu.VMEM((320,), jnp.int32),
        pltpu.VMEM((320,), jnp.float32),
        pltpu.VMEM((160,), jnp.int32),
        pltpu.VMEM((160, C), jnp.float32),
        pltpu.SemaphoreType.DMA,
    ],
)
def _sc_pre(xf, pidxf, seedt, bids, nids, emb,
            augf_o, seedg_o, shal_o,
            aidx_v, abuf_v, sidx_v, sbuf_v, nidx_v, ebuf_v, sem):
    c = lax.axis_index("c")
    s = lax.axis_index("s")
    w = c * NS + s
    r0 = jnp.minimum(ROWS_PER_W * w, N - ROWS_PER_W)

    for p in range(2):
        pltpu.sync_copy(pidxf.at[pl.ds((r0 + 160 * p) * C, 20480)], aidx_v)
        cps = [
            pltpu.async_copy(
                xf.at[aidx_v.at[pl.ds(2048 * g, 2048)]],
                abuf_v.at[pl.ds(2048 * g, 2048)], sem)
            for g in range(10)
        ]
        for cp in cps:
            cp.wait()
        pltpu.sync_copy(abuf_v, augf_o.at[pl.ds((r0 + 160 * p) * C, 20480)])

    pltpu.sync_copy(bids.at[pl.ds(r0, 320)], sidx_v)
    pltpu.async_copy(seedt.at[sidx_v], sbuf_v, sem).wait()
    pltpu.sync_copy(sbuf_v, seedg_o.at[pl.ds(r0, 320)])

    for p in range(2):
        pltpu.sync_copy(nids.at[pl.ds(r0 + 160 * p, 160)], nidx_v)
        pltpu.async_copy(emb.at[nidx_v], ebuf_v, sem).wait()
        pltpu.sync_copy(ebuf_v, shal_o.at[pl.ds(r0 + 160 * p, 160)])


def _b_body(x_b, aug_b, sg_b, nt_b, shal_b, wenc, benc, wtime, btime, freqs_b,
            hpre_o, augpre_o):
    wenc_m = wenc[...]
    base = jnp.dot(x_b[...], wenc_m, preferred_element_type=jnp.float32)
    aug = jnp.dot(aug_b[...], wenc_m, preferred_element_type=jnp.float32)
    rel = sg_b[...] - nt_b[...]
    feats = jnp.cos(rel * freqs_b[...])
    tfeat = jnp.dot(feats, wtime[...], preferred_element_type=jnp.float32)
    add = tfeat + benc[...] + btime[...] + shal_b[...]
    hpre_o[...] = base + add
    augpre_o[...] = aug + add


def _tc_pre(x, aug, seedg, ntime, shallow, wenc, benc, wtime, btime):
    blk = N // 10
    return pl.pallas_call(
        _b_body,
        grid=(10,),
        in_specs=[
            pl.BlockSpec((blk, C), lambda i: (i, 0)),
            pl.BlockSpec((blk, C), lambda i: (i, 0)),
            pl.BlockSpec((blk, 1), lambda i: (i, 0)),
            pl.BlockSpec((blk, 1), lambda i: (i, 0)),
            pl.BlockSpec((blk, C), lambda i: (i, 0)),
            pl.BlockSpec((C, C), lambda i: (0, 0)),
            pl.BlockSpec((1, C), lambda i: (0, 0)),
            pl.BlockSpec((C, C), lambda i: (0, 0)),
            pl.BlockSpec((1, C), lambda i: (0, 0)),
            pl.BlockSpec((1, C), lambda i: (0, 0)),
        ],
        out_specs=[
            pl.BlockSpec((blk, C), lambda i: (i, 0)),
            pl.BlockSpec((blk, C), lambda i: (i, 0)),
        ],
        out_shape=[
            jax.ShapeDtypeStruct((N, C), jnp.float32),
            jax.ShapeDtypeStruct((N, C), jnp.float32),
        ],
    )(x, aug, seedg, ntime, shallow, wenc, benc, wtime, btime,
      jnp.asarray(_FREQS).reshape(1, C))


BLKS = 2560
EP = BLKS * 128
CHKS = BLKS // 16
CPT = CHKS // NS
RPT = NP // NS


@functools.partial(
    pl.kernel,
    out_type=[
        jax.ShapeDtypeStruct((NP, C), jnp.float32),
        jax.ShapeDtypeStruct((NP, C), jnp.float32),
        jax.ShapeDtypeStruct((NP,), jnp.float32),
    ],
    mesh=_SC_MESH,
    scratch_types=[
        pltpu.VMEM((128, C), jnp.float32),
        pltpu.VMEM((128, C), jnp.float32),
        pltpu.VMEM((16, 128), jnp.int32),
        pltpu.VMEM((16, 128), jnp.int32),
        pltpu.VMEM((128,), jnp.float32),
        pltpu.VMEM((RPT,), jnp.float32),
        pltpu.VMEM_SHARED((NP, C), jnp.float32),
        pltpu.VMEM_SHARED((NP,), jnp.float32),
        pltpu.SemaphoreType.DMA,
        pltpu.SemaphoreType.DMA,
        pltpu.SemaphoreType.DMA,
        pltpu.SemaphoreType.DMA,
        pltpu.SemaphoreType.DMA,
    ],
)
def _sc_agg(hpre, augpre, src2d, dst2d,
            aggh_o, aggaug_o, cnt_o,
            rows0_v, rows1_v, sidx_v, didx_v, ones_v, cbuf_v,
            agg_sh, cnt_sh, sem0, sem1, ssem0, ssem1, csem):
    c = lax.axis_index("c")
    s = lax.axis_index("s")

    def zrow(i, carry):
        def zj(j, inner):
            rows0_v[i, pl.ds(16 * j, 16)] = jnp.zeros((16,), jnp.float32)
            return inner
        return lax.fori_loop(0, C // 16, zj, carry)

    lax.fori_loop(0, 128, zrow, 0)

    def zcb(i, carry):
        cbuf_v[pl.ds(16 * i, 16)] = jnp.zeros((16,), jnp.float32)
        return carry

    lax.fori_loop(0, RPT // 16, zcb, 0)

    def of(i, carry):
        ones_v[pl.ds(16 * i, 16)] = jnp.ones((16,), jnp.float32)
        return carry

    lax.fori_loop(0, 8, of, 0)

    for k in range(RPT // 128):
        pltpu.sync_copy(rows0_v, agg_sh.at[pl.ds(RPT * s + 128 * k, 128)])
    pltpu.sync_copy(cbuf_v, cnt_sh.at[pl.ds(RPT * s, RPT)])
    plsc.subcore_barrier()

    def _edges(tbl, do_cnt):
        def estep(t, carry):
            q = s + NS * t
            pltpu.sync_copy(src2d.at[pl.ds(16 * q, 16)], sidx_v)
            pltpu.sync_copy(dst2d.at[pl.ds(16 * q, 16)], didx_v)
            bufs = (rows0_v, rows1_v)
            gsems = (sem0, sem1)
            ssems = (ssem0, ssem1)
            def _gather2(j, b):
                return [
                    pltpu.async_copy(
                        tbl.at[sidx_v.at[j, pl.ds(64 * h, 64)]],
                        bufs[b].at[pl.ds(64 * h, 64)], gsems[b])
                    for h in range(2)
                ]

            cp_prev = _gather2(0, 0)
            sc_prev = [None, None]
            ccp = None
            for j in range(16):
                if j + 1 < 16:
                    b = (j + 1) % 2
                    if sc_prev[b] is not None:
                        sc_prev[b].wait()
                        sc_prev[b] = None
                    cp_next = _gather2(j + 1, b)
                for cp in cp_prev:
                    cp.wait()
                sc_prev[j % 2] = pltpu.async_copy(
                    bufs[j % 2], agg_sh.at[didx_v.at[j]], ssems[j % 2],
                    add=True)
                if do_cnt:
                    if ccp is not None:
                        ccp.wait()
                    ccp = pltpu.async_copy(ones_v, cnt_sh.at[didx_v.at[j]],
                                           csem, add=True)
                if j + 1 < 16:
                    cp_prev = cp_next
            for b in range(2):
                if sc_prev[b] is not None:
                    sc_prev[b].wait()
            if ccp is not None:
                ccp.wait()
            return carry

        lax.fori_loop(0, CPT, estep, 0)

    @pl.when(c == 0)
    def _ch0():
        _edges(hpre, True)

    @pl.when(c == 1)
    def _ch1():
        _edges(augpre, False)

    plsc.subcore_barrier()

    def _write_agg(out):
        for k in range(RPT // 128):
            buf = rows0_v if k % 2 == 0 else rows1_v
            pltpu.sync_copy(agg_sh.at[pl.ds(RPT * s + 128 * k, 128)], buf)
            pltpu.sync_copy(buf, out.at[pl.ds(RPT * s + 128 * k, 128)])

    @pl.when(c == 0)
    def _w0():
        _write_agg(aggh_o)
        pltpu.sync_copy(cnt_sh.at[pl.ds(RPT * s, RPT)], cbuf_v)
        pltpu.sync_copy(cbuf_v, cnt_o.at[pl.ds(RPT * s, RPT)])

    @pl.when(c == 1)
    def _w1():
        _write_agg(aggaug_o)


def _d_body(hp, ap, ah, aa, cnt, wself, wneigh, bgnn, whead, bhead,
            wpred, bpred, out_o, augproj_o, hproj_o):
    inv = 1.0 / jnp.maximum(cnt[...], 1.0)
    ws = wself[...]
    wn = wneigh[...]
    hg = jnp.maximum(
        jnp.dot(hp[...], ws, preferred_element_type=jnp.float32)
        + jnp.dot(ah[...] * inv, wn, preferred_element_type=jnp.float32)
        + bgnn[...], 0.0)
    ag = jnp.maximum(
        jnp.dot(ap[...], ws, preferred_element_type=jnp.float32)
        + jnp.dot(aa[...] * inv, wn, preferred_element_type=jnp.float32)
        + bgnn[...], 0.0)
    out_o[...] = jnp.dot(ag, wpred[...],
                         preferred_element_type=jnp.float32) + bpred[...]
    augproj_o[...] = jnp.dot(ag, whead[...],
                             preferred_element_type=jnp.float32) + bhead[...]
    hproj_o[...] = jnp.dot(hg, whead[...],
                           preferred_element_type=jnp.float32) + bhead[...]


def _tc_post(hpre, augpre, aggh, aggaug, cnt, wself, wneigh, bgnn2,
             whead, bhead2, wpred, bpred2):
    blk = N // 10
    return pl.pallas_call(
        _d_body,
        grid=(10,),
        in_specs=[
            pl.BlockSpec((blk, C), lambda i: (i, 0)),
            pl.BlockSpec((blk, C), lambda i: (i, 0)),
            pl.BlockSpec((blk, C), lambda i: (i, 0)),
            pl.BlockSpec((blk, C), lambda i: (i, 0)),
            pl.BlockSpec((blk, 1), lambda i: (i, 0)),
            pl.BlockSpec((C, C), lambda i: (0, 0)),
            pl.BlockSpec((C, C), lambda i: (0, 0)),
            pl.BlockSpec((1, C), lambda i: (0, 0)),
            pl.BlockSpec((C, HD), lambda i: (0, 0)),
            pl.BlockSpec((1, HD), lambda i: (0, 0)),
            pl.BlockSpec((C, OC), lambda i: (0, 0)),
            pl.BlockSpec((1, OC), lambda i: (0, 0)),
        ],
        out_specs=[
            pl.BlockSpec((blk, OC), lambda i: (i, 0)),
            pl.BlockSpec((blk, HD), lambda i: (i, 0)),
            pl.BlockSpec((blk, HD), lambda i: (i, 0)),
        ],
        out_shape=[
            jax.ShapeDtypeStruct((N, OC), jnp.float32),
            jax.ShapeDtypeStruct((N, HD), jnp.float32),
            jax.ShapeDtypeStruct((N, HD), jnp.float32),
        ],
    )(hpre, augpre, aggh, aggaug, cnt, wself, wneigh, bgnn2, whead, bhead2,
      wpred, bpred2)


def kernel(x, edge_index, seed_time, node_time, batch_ids, n_id,
           W_enc, b_enc, W_time, b_time, emb_table,
           W_self, W_neigh, b_gnn, W_head, b_head, W_pred, b_pred):
    pidx2d = jnp.asarray(_PIDX2D) if _PIDX2D is not None else _aug_pidx2d()
    aug_f, seedg, shallow = _sc_pre(
        x.reshape(-1), pidx2d.reshape(-1), seed_time, batch_ids, n_id,
        emb_table)
    h_pre, aug_pre = _tc_pre(
        x, aug_f.reshape(N, C), seedg.reshape(N, 1), node_time.reshape(N, 1),
        shallow, W_enc, b_enc.reshape(1, C), W_time, b_time.reshape(1, C))
    spad = jnp.zeros((EP - E,), jnp.int32)
    dpad = jnp.full((EP - E,), NP - 1, jnp.int32)
    src2d = jnp.concatenate([edge_index[0], spad]).reshape(BLKS, 128)
    dst2d = jnp.concatenate([edge_index[1], dpad]).reshape(BLKS, 128)
    agg_h, agg_aug, cnt = _sc_agg(h_pre, aug_pre, src2d, dst2d)
    return _tc_post(
        h_pre, aug_pre, agg_h[:N], agg_aug[:N], cnt[:N].reshape(N, 1),
        W_self, W_neigh,
        b_gnn.reshape(1, C), W_head, b_head.reshape(1, HD),
        W_pred, b_pred.reshape(1, OC))
